# baseline (speedup 1.0000x reference)
import functools

import jax
import jax.numpy as jnp
from jax import lax
from jax.experimental import pallas as pl
from jax.experimental.pallas import tpu as pltpu
from jax.experimental.pallas import tpu_sc as plsc

NC = 2
NS = 16
NW = NC * NS
NP = 50176
E_TOT = 800000
EPT = E_TOT // NS
EW = 2000
NWIN = EPT // EW
RPT = NP // NS
R = 3136
N_RB = NP // R
SEGP = 528
RW = NP // NW
WR = RW // 2

ED = E_TOT // NW
EWD = 1000
NWD = ED // EWD


def _deg_body(dst_ref, out_ref, ones_v, didx_v, zero_v, acc_sp):
    c = lax.axis_index("c")
    s = lax.axis_index("s")

    def fill(i, _):
        ones_v[pl.ds(i * 16, 16)] = jnp.full((16,), 1.0, jnp.float32)
        return 0
    lax.fori_loop(0, EWD // 16, fill, 0)

    def zfill(i, _):
        zero_v[pl.ds(i * 16, 16)] = jnp.zeros((16,), jnp.float32)
        return 0
    lax.fori_loop(0, RPT // 16, zfill, 0)
    pltpu.sync_copy(zero_v, acc_sp.at[pl.ds(s * RPT, RPT)])
    plsc.subcore_barrier()

    ebase = (c * NS + s) * ED

    def win(w, _):
        pltpu.sync_copy(dst_ref.at[pl.ds(ebase + w * EWD, EWD)], didx_v)
        pltpu.sync_copy(ones_v, acc_sp.at[didx_v], add=True)
        return 0
    lax.fori_loop(0, NWD, win, 0)
    plsc.subcore_barrier()
    pltpu.sync_copy(acc_sp.at[pl.ds(s * RPT, RPT)],
                    out_ref.at[c, pl.ds(s * RPT, RPT)])


def _sc_deg(dst):
    mesh = plsc.VectorSubcoreMesh(core_axis_name="c", subcore_axis_name="s")
    return pl.kernel(
        _deg_body,
        out_type=jax.ShapeDtypeStruct((NC, NP), jnp.float32),
        mesh=mesh,
        compiler_params=pltpu.CompilerParams(use_tc_tiling_on_sc=False),
        scratch_types=[
            pltpu.VMEM((EWD,), jnp.float32),
            pltpu.VMEM((EWD,), jnp.int32),
            pltpu.VMEM((RPT,), jnp.float32),
            pltpu.VMEM_SHARED((NP,), jnp.float32),
        ],
    )(dst)


E_EXT = 864000
EPTX = E_EXT // NS
NWINX = EPTX // EW


def _scatter_body(nf, y_ref, zeros_ref, src_ref, dst_ref, out_ref,
                  idx_a, didx_a, rows_a, idx_b, didx_b, rows_b,
                  acc_sp, gsem_a, gsem_b):
    c = lax.axis_index("c")
    s = lax.axis_index("s")
    ebase = s * EPTX

    def chunk_body(jj, _):
        j = c + 2 * jj
        pltpu.sync_copy(zeros_ref.at[pl.ds(s * RPT, RPT)],
                        acc_sp.at[pl.ds(s * RPT, RPT)])
        plsc.subcore_barrier()

        yj = y_ref.at[j]

        def load(w, idx_v, didx_v):
            base = ebase + w * EW
            pltpu.sync_copy(src_ref.at[pl.ds(base, EW)], idx_v)
            pltpu.sync_copy(dst_ref.at[pl.ds(base, EW)], didx_v)

        load(0, idx_a, didx_a)
        pltpu.async_copy(yj.at[idx_a], rows_a, gsem_a)

        def pair(k, _):
            w = 2 * k
            load(w + 1, idx_b, didx_b)
            pltpu.async_copy(yj.at[idx_b], rows_b, gsem_b)
            pltpu.make_async_copy(yj.at[idx_a], rows_a, gsem_a).wait()
            pltpu.sync_copy(rows_a, acc_sp.at[didx_a], add=True)
            load(w + 2, idx_a, didx_a)
            pltpu.async_copy(yj.at[idx_a], rows_a, gsem_a)
            pltpu.make_async_copy(yj.at[idx_b], rows_b, gsem_b).wait()
            pltpu.sync_copy(rows_b, acc_sp.at[didx_b], add=True)
            return 0

        lax.fori_loop(0, (NWINX - 1) // 2, pair, 0)
        pltpu.make_async_copy(yj.at[idx_a], rows_a, gsem_a).wait()
        pltpu.sync_copy(rows_a, acc_sp.at[didx_a], add=True)

        plsc.subcore_barrier()
        pltpu.sync_copy(acc_sp.at[pl.ds(s * RPT, RPT)],
                        out_ref.at[pl.ds(s * RPT, RPT), pl.ds(j * 16, 16)])
        return 0

    lax.fori_loop(0, (nf - c + 1) // 2, chunk_body, 0)


def _sc_scatter(y_t, zeros_hbm, src, dst):
    nf = y_t.shape[0]
    mesh = plsc.VectorSubcoreMesh(core_axis_name="c", subcore_axis_name="s")
    return pl.kernel(
        functools.partial(_scatter_body, nf),
        out_type=jax.ShapeDtypeStruct((NP, nf * 16), jnp.float32),
        mesh=mesh,
        compiler_params=pltpu.CompilerParams(use_tc_tiling_on_sc=False),
        scratch_types=[
            pltpu.VMEM((EW,), jnp.int32),
            pltpu.VMEM((EW,), jnp.int32),
            pltpu.VMEM((EW, 16), jnp.float32),
            pltpu.VMEM((EW,), jnp.int32),
            pltpu.VMEM((EW,), jnp.int32),
            pltpu.VMEM((EW, 16), jnp.float32),
            pltpu.VMEM_SHARED((NP, 16), jnp.float32),
            pltpu.SemaphoreType.DMA,
            pltpu.SemaphoreType.DMA,
        ],
    )(y_t, zeros_hbm, src, dst)


def _segmax_body(nf, s3_ref, dis_ref, b_ref, batch_ref, out_ref,
                 rows_a, rows_b, batch_v, dis_v, b_v, out_v,
                 gsem_a, gsem_b):
    c = lax.axis_index("c")
    s = lax.axis_index("s")
    w = c * NS + s
    rbase = w * RW

    pltpu.sync_copy(batch_ref.at[pl.ds(rbase, RW)], batch_v.at[pl.ds(0, RW)])
    pltpu.sync_copy(dis_ref.at[pl.ds(rbase, RW)], dis_v.at[pl.ds(0, RW)])
    pltpu.sync_copy(b_ref, b_v)

    def ifill(i, _):
        out_v[pl.ds(i * 16, 16)] = jnp.full((16,), -1e30, jnp.float32)
        return 0
    lax.fori_loop(0, SEGP, ifill, 0)
    b_first = batch_v[pl.ds(0, 16)][0]
    b_last = batch_v[pl.ds(RW - 16, 16)][15]

    def chunk_body(j, _):
        lax.fori_loop(b_first, b_last + 1, ifill, 0)
        bj = b_v[pl.ds(j * 16, 16)]
        pltpu.async_copy(
            s3_ref.at[pl.ds(rbase, WR), pl.ds(j * 16, 16)], rows_a, gsem_a)
        pltpu.async_copy(
            s3_ref.at[pl.ds(rbase + WR, WR), pl.ds(j * 16, 16)], rows_b, gsem_b)

        def scan_rows(rows_v, roff, carry):
            def row_body(r, car):
                bprev, m = car
                b = batch_v[pl.ds(roff + r, 16)][0]
                d = dis_v[pl.ds(roff + r, 16)][0]
                row = jnp.maximum(rows_v[r] * d + bj, 0.0)

                @pl.when(b != bprev)
                def _():
                    out_v[pl.ds(bprev * 16, 16)] = m

                m = jnp.where(b != bprev, jnp.full((16,), -1e30, jnp.float32), m)
                return (b, jnp.maximum(m, row))
            return lax.fori_loop(0, WR, row_body, carry)

        pltpu.make_async_copy(
            s3_ref.at[pl.ds(rbase, WR), pl.ds(j * 16, 16)], rows_a, gsem_a).wait()
        carry = scan_rows(rows_a, 0,
                          (batch_v[pl.ds(0, 16)][0],
                           jnp.full((16,), -1e30, jnp.float32)))
        pltpu.make_async_copy(
            s3_ref.at[pl.ds(rbase + WR, WR), pl.ds(j * 16, 16)], rows_b, gsem_b).wait()
        bprev, m = scan_rows(rows_b, WR, carry)
        out_v[pl.ds(bprev * 16, 16)] = m

        pltpu.sync_copy(out_v, out_ref.at[w, pl.ds(j * SEGP * 16, SEGP * 16)])
        return 0

    lax.fori_loop(0, nf, chunk_body, 0)


def _sc_segmax(s3, dis1d, b3, batch_pad):
    nf = s3.shape[1] // 16
    mesh = plsc.VectorSubcoreMesh(core_axis_name="c", subcore_axis_name="s")
    return pl.kernel(
        functools.partial(_segmax_body, nf),
        out_type=jax.ShapeDtypeStruct((NW, nf * SEGP * 16), jnp.float32),
        mesh=mesh,
        compiler_params=pltpu.CompilerParams(use_tc_tiling_on_sc=False),
        scratch_types=[
            pltpu.VMEM((WR, 16), jnp.float32),
            pltpu.VMEM((WR, 16), jnp.float32),
            pltpu.VMEM((RW + 16,), jnp.int32),
            pltpu.VMEM((RW + 16,), jnp.float32),
            pltpu.VMEM((nf * 16,), jnp.float32),
            pltpu.VMEM((SEGP * 16,), jnp.float32),
            pltpu.SemaphoreType.DMA,
            pltpu.SemaphoreType.DMA,
        ],
    )(s3, dis1d, b3, batch_pad)


def _dis_body(p_ref, o_ref):
    o_ref[...] = 1.0 / jnp.sqrt(1.0 + p_ref[0] + p_ref[1])


def _tc_dis(partials):
    p = partials.reshape(NC, NP // 128, 128)
    out = pl.pallas_call(
        _dis_body,
        out_shape=jax.ShapeDtypeStruct((NP // 128, 128), jnp.float32),
    )(p)
    return out.reshape(NP, 1)


def _l1_body(x_ref, w_ref, dis_ref, o_ref):
    o_ref[...] = (jnp.dot(x_ref[...], w_ref[0],
                          preferred_element_type=jnp.float32)
                  * dis_ref[...])[None]


def _tc_layer1(x_pad, W, dis):
    k = x_pad.shape[1]
    nf = W.shape[1] // 16
    w_r = W.reshape(k, nf, 16).transpose(1, 0, 2)
    return pl.pallas_call(
        _l1_body,
        grid=(N_RB, nf),
        in_specs=[
            pl.BlockSpec((R, k), lambda i, j: (i, 0)),
            pl.BlockSpec((1, k, 16), lambda i, j: (j, 0, 0)),
            pl.BlockSpec((R, 1), lambda i, j: (i, 0)),
        ],
        out_specs=pl.BlockSpec((1, R, 16), lambda i, j: (j, i, 0)),
        out_shape=jax.ShapeDtypeStruct((nf, NP, 16), jnp.float32),
    )(x_pad, w_r, dis)


def _mid_body(s_ref, w_ref, dis_ref, b_ref, o_ref, h_scr):
    @pl.when(pl.program_id(1) == 0)
    def _():
        h_scr[...] = jnp.maximum(s_ref[...] * dis_ref[...] + b_ref[...], 0.0)

    o_ref[...] = (jnp.dot(h_scr[...], w_ref[0],
                          preferred_element_type=jnp.float32)
                  * dis_ref[...])[None]


def _tc_mid(s_prev, W, dis, b_prev):
    fin = s_prev.shape[1]
    nf = W.shape[1] // 16
    w_r = W.reshape(fin, nf, 16).transpose(1, 0, 2)
    return pl.pallas_call(
        _mid_body,
        grid=(N_RB, nf),
        in_specs=[
            pl.BlockSpec((R, fin), lambda i, j: (i, 0)),
            pl.BlockSpec((1, fin, 16), lambda i, j: (j, 0, 0)),
            pl.BlockSpec((R, 1), lambda i, j: (i, 0)),
            pl.BlockSpec((1, fin), lambda i, j: (0, 0)),
        ],
        out_specs=pl.BlockSpec((1, R, 16), lambda i, j: (j, i, 0)),
        out_shape=jax.ShapeDtypeStruct((nf, NP, 16), jnp.float32),
        scratch_shapes=[pltpu.VMEM((R, fin), jnp.float32)],
    )(s_prev, w_r, dis, b_prev)


def _gmax_body(nf, p_ref, o_ref):
    @pl.when(pl.program_id(0) == 0)
    def _():
        o_ref[...] = jnp.full_like(o_ref, -1e30)
    g = jnp.concatenate([p_ref[0, j] for j in range(nf)], axis=1)
    o_ref[...] = jnp.maximum(o_ref[...], g)


def _tc_gmax(partials_flat, nf):
    p = partials_flat.reshape(NW, nf, SEGP, 16)
    return pl.pallas_call(
        functools.partial(_gmax_body, nf),
        grid=(NW,),
        in_specs=[pl.BlockSpec((1, nf, SEGP, 16), lambda i: (i, 0, 0, 0))],
        out_specs=pl.BlockSpec((SEGP, nf * 16), lambda i: (0, 0)),
        out_shape=jax.ShapeDtypeStruct((SEGP, nf * 16), jnp.float32),
    )(p)


def _head_body(g_ref, tp_ref, wg1_ref, bg1_ref, wg2_ref, bg2_ref,
               wf1_ref, bf1_ref, wf2_ref, bf2_ref, wo_ref, bo_ref, out_ref):
    g = g_ref[...]
    h = jnp.maximum(jnp.dot(g, wg1_ref[...], preferred_element_type=jnp.float32)
                    + bg1_ref[...], 0.0)
    g2 = jnp.dot(h, wg2_ref[...], preferred_element_type=jnp.float32) + bg2_ref[...]
    xc = jnp.concatenate([g2, tp_ref[...]], axis=1)
    f1 = jnp.maximum(jnp.dot(xc, wf1_ref[...], preferred_element_type=jnp.float32)
                     + bf1_ref[...], 0.0)
    f2 = jnp.maximum(jnp.dot(f1, wf2_ref[...], preferred_element_type=jnp.float32)
                     + bf2_ref[...], 0.0)
    out_ref[...] = jnp.dot(f2, wo_ref[...], preferred_element_type=jnp.float32) + bo_ref[...]


def _mlp_head(g, T, P, Wg1, bg1, Wg2, bg2, Wf1, bf1, Wf2, bf2, Wo, bo):
    B = T.shape[0]
    F = Wg1.shape[0]
    Fp = g.shape[1]
    wg1 = jnp.pad(Wg1, ((0, Fp - F), (0, 0)))
    tp = jnp.pad(jnp.stack([T, P], axis=1), ((0, 0), (0, 126)))
    wf1 = jnp.pad(Wf1, ((0, 256 - Wf1.shape[0]), (0, 0)))
    wo = jnp.pad(Wo, ((0, 0), (0, 127)))
    bo_p = jnp.pad(bo, ((0, 127)))
    out = pl.pallas_call(
        _head_body,
        out_shape=jax.ShapeDtypeStruct((B, 128), jnp.float32),
    )(g, tp, wg1, bg1.reshape(1, -1), Wg2, bg2.reshape(1, -1),
      wf1, bf1.reshape(1, -1), Wf2, bf2.reshape(1, -1), wo, bo_p.reshape(1, -1))
    return out[:, :1]


def _pad_w(W, b):
    fin, fout = W.shape
    fi = ((fin + 15) // 16) * 16
    fo = ((fout + 15) // 16) * 16
    return (jnp.pad(W, ((0, fi - fin), (0, fo - fout))),
            jnp.pad(b, (0, fo - fout)).reshape(1, fo))


def kernel(x, edge_index, batch, T, P, W1, b1, W2, b2, W3, b3,
           Wg1, bg1, Wg2, bg2, Wf1, bf1, Wf2, bf2, Wo, bo):
    n = x.shape[0]
    B = T.shape[0]
    src = edge_index[0]
    dst = edge_index[1]

    partials = _sc_deg(dst)
    dis = _tc_dis(partials)

    w1, b1p = _pad_w(W1, b1)
    x_pad = jnp.pad(x, ((0, NP - n), (0, w1.shape[0] - x.shape[1])))
    w2, b2p = _pad_w(W2, b2)
    w3, b3p = _pad_w(W3, b3)
    batch_pad = jnp.concatenate([batch, jnp.full((NP - n,), B, jnp.int32)])

    iota = jnp.arange(n, dtype=jnp.int32)
    pad_e = jnp.full((E_EXT - E_TOT - n,), n, jnp.int32)
    src_ext = jnp.concatenate([src, iota, pad_e])
    dst_ext = jnp.concatenate([dst, iota, pad_e])
    zeros_hbm = jnp.zeros((NP, 16), jnp.float32)

    y1 = _tc_layer1(x_pad, w1, dis)
    s1 = _sc_scatter(y1, zeros_hbm, src_ext, dst_ext)
    y2 = _tc_mid(s1, w2, dis, b1p)
    s2 = _sc_scatter(y2, zeros_hbm, src_ext, dst_ext)
    y3 = _tc_mid(s2, w3, dis, b2p)
    s3 = _sc_scatter(y3, zeros_hbm, src_ext, dst_ext)

    partials_g = _sc_segmax(s3, dis.reshape(NP), b3p.reshape(-1), batch_pad)
    g = _tc_gmax(partials_g, w3.shape[1] // 16)
    return _mlp_head(g[:B, :], T, P, Wg1, bg1, Wg2, bg2, Wf1, bf1, Wf2, bf2, Wo, bo)


# speedup vs baseline: 1.3220x; 1.3220x over previous
import functools

import jax
import jax.numpy as jnp
from jax import lax
from jax.experimental import pallas as pl
from jax.experimental.pallas import tpu as pltpu
from jax.experimental.pallas import tpu_sc as plsc

NC = 2
NS = 16
NW = NC * NS
NP = 50176
E_TOT = 800000
EPT = E_TOT // NS
EW = 2000
NWIN = EPT // EW
RPT = NP // NS
R = 3136
N_RB = NP // R
SEGP = 528
RW = NP // NW
WR = RW // 2

ED = E_TOT // NW
EWD = 1000
NWD = ED // EWD


def _deg_body(dst_ref, out_ref, ones_v, didx_v, zero_v, acc_sp):
    c = lax.axis_index("c")
    s = lax.axis_index("s")

    def fill(i, _):
        ones_v[pl.ds(i * 16, 16)] = jnp.full((16,), 1.0, jnp.float32)
        return 0
    lax.fori_loop(0, EWD // 16, fill, 0)

    def zfill(i, _):
        zero_v[pl.ds(i * 16, 16)] = jnp.zeros((16,), jnp.float32)
        return 0
    lax.fori_loop(0, RPT // 16, zfill, 0)
    pltpu.sync_copy(zero_v, acc_sp.at[pl.ds(s * RPT, RPT)])
    plsc.subcore_barrier()

    ebase = (c * NS + s) * ED

    def win(w, _):
        pltpu.sync_copy(dst_ref.at[pl.ds(ebase + w * EWD, EWD)], didx_v)
        pltpu.sync_copy(ones_v, acc_sp.at[didx_v], add=True)
        return 0
    lax.fori_loop(0, NWD, win, 0)
    plsc.subcore_barrier()
    pltpu.sync_copy(acc_sp.at[pl.ds(s * RPT, RPT)],
                    out_ref.at[c, pl.ds(s * RPT, RPT)])


def _sc_deg(dst):
    mesh = plsc.VectorSubcoreMesh(core_axis_name="c", subcore_axis_name="s")
    return pl.kernel(
        _deg_body,
        out_type=jax.ShapeDtypeStruct((NC, NP), jnp.float32),
        mesh=mesh,
        compiler_params=pltpu.CompilerParams(use_tc_tiling_on_sc=False),
        scratch_types=[
            pltpu.VMEM((EWD,), jnp.float32),
            pltpu.VMEM((EWD,), jnp.int32),
            pltpu.VMEM((RPT,), jnp.float32),
            pltpu.VMEM_SHARED((NP,), jnp.float32),
        ],
    )(dst)


E_EXT = 864000
EPTX = E_EXT // NS
NWINX = EPTX // EW


def _scatter_body(nf, y_ref, zeros_ref, src_ref, dst_ref, out_ref,
                  idx_a, didx_a, rows_a, idx_b, didx_b, rows_b,
                  acc_sp, gsem_a, gsem_b):
    c = lax.axis_index("c")
    s = lax.axis_index("s")
    ebase = s * EPTX

    def chunk_body(jj, _):
        j = c + 2 * jj
        pltpu.sync_copy(zeros_ref.at[pl.ds(s * RPT, RPT)],
                        acc_sp.at[pl.ds(s * RPT, RPT)])
        plsc.subcore_barrier()

        yj = y_ref.at[j]

        def load(w, idx_v, didx_v):
            base = ebase + w * EW
            pltpu.sync_copy(src_ref.at[pl.ds(base, EW)], idx_v)
            pltpu.sync_copy(dst_ref.at[pl.ds(base, EW)], didx_v)

        load(0, idx_a, didx_a)
        pltpu.async_copy(yj.at[idx_a], rows_a, gsem_a)

        def pair(k, _):
            w = 2 * k
            load(w + 1, idx_b, didx_b)
            pltpu.async_copy(yj.at[idx_b], rows_b, gsem_b)
            pltpu.make_async_copy(yj.at[idx_a], rows_a, gsem_a).wait()
            pltpu.sync_copy(rows_a, acc_sp.at[didx_a], add=True)
            load(w + 2, idx_a, didx_a)
            pltpu.async_copy(yj.at[idx_a], rows_a, gsem_a)
            pltpu.make_async_copy(yj.at[idx_b], rows_b, gsem_b).wait()
            pltpu.sync_copy(rows_b, acc_sp.at[didx_b], add=True)
            return 0

        lax.fori_loop(0, (NWINX - 1) // 2, pair, 0)
        pltpu.make_async_copy(yj.at[idx_a], rows_a, gsem_a).wait()
        pltpu.sync_copy(rows_a, acc_sp.at[didx_a], add=True)

        plsc.subcore_barrier()
        pltpu.sync_copy(acc_sp.at[pl.ds(s * RPT, RPT)],
                        out_ref.at[pl.ds(s * RPT, RPT), pl.ds(j * 16, 16)])
        return 0

    lax.fori_loop(0, (nf - c + 1) // 2, chunk_body, 0)


def _sc_scatter(y_t, zeros_hbm, src, dst):
    nf = y_t.shape[0]
    mesh = plsc.VectorSubcoreMesh(core_axis_name="c", subcore_axis_name="s")
    return pl.kernel(
        functools.partial(_scatter_body, nf),
        out_type=jax.ShapeDtypeStruct((NP, nf * 16), jnp.float32),
        mesh=mesh,
        compiler_params=pltpu.CompilerParams(use_tc_tiling_on_sc=False),
        scratch_types=[
            pltpu.VMEM((EW,), jnp.int32),
            pltpu.VMEM((EW,), jnp.int32),
            pltpu.VMEM((EW, 16), jnp.float32),
            pltpu.VMEM((EW,), jnp.int32),
            pltpu.VMEM((EW,), jnp.int32),
            pltpu.VMEM((EW, 16), jnp.float32),
            pltpu.VMEM_SHARED((NP, 16), jnp.float32),
            pltpu.SemaphoreType.DMA,
            pltpu.SemaphoreType.DMA,
        ],
    )(y_t, zeros_hbm, src, dst)


def _segmax_body(nf, s3_ref, dis_ref, b_ref, batch_ref, out_ref,
                 rows_a, rows_b, batch_v, dis_v, b_v, out_v,
                 gsem_a, gsem_b):
    c = lax.axis_index("c")
    s = lax.axis_index("s")
    w = c * NS + s
    rbase = w * RW

    pltpu.sync_copy(batch_ref.at[pl.ds(rbase, RW)], batch_v.at[pl.ds(0, RW)])
    pltpu.sync_copy(dis_ref.at[pl.ds(rbase, RW)], dis_v.at[pl.ds(0, RW)])
    pltpu.sync_copy(b_ref, b_v)

    def ifill(i, _):
        out_v[pl.ds(i * 16, 16)] = jnp.full((16,), -1e30, jnp.float32)
        return 0
    lax.fori_loop(0, SEGP, ifill, 0)
    b_first = batch_v[pl.ds(0, 16)][0]
    b_last = batch_v[pl.ds(RW - 16, 16)][15]

    def chunk_body(j, _):
        lax.fori_loop(b_first, b_last + 1, ifill, 0)
        bj = b_v[pl.ds(j * 16, 16)]
        pltpu.async_copy(
            s3_ref.at[pl.ds(rbase, WR), pl.ds(j * 16, 16)], rows_a, gsem_a)
        pltpu.async_copy(
            s3_ref.at[pl.ds(rbase + WR, WR), pl.ds(j * 16, 16)], rows_b, gsem_b)

        def scan_rows(rows_v, roff, carry):
            def row_body(r, car):
                bprev, m = car
                b = batch_v[pl.ds(roff + r, 16)][0]
                d = dis_v[pl.ds(roff + r, 16)][0]
                row = jnp.maximum(rows_v[r] * d + bj, 0.0)

                @pl.when(b != bprev)
                def _():
                    out_v[pl.ds(bprev * 16, 16)] = m

                m = jnp.where(b != bprev, jnp.full((16,), -1e30, jnp.float32), m)
                return (b, jnp.maximum(m, row))
            return lax.fori_loop(0, WR, row_body, carry)

        pltpu.make_async_copy(
            s3_ref.at[pl.ds(rbase, WR), pl.ds(j * 16, 16)], rows_a, gsem_a).wait()
        carry = scan_rows(rows_a, 0,
                          (batch_v[pl.ds(0, 16)][0],
                           jnp.full((16,), -1e30, jnp.float32)))
        pltpu.make_async_copy(
            s3_ref.at[pl.ds(rbase + WR, WR), pl.ds(j * 16, 16)], rows_b, gsem_b).wait()
        bprev, m = scan_rows(rows_b, WR, carry)
        out_v[pl.ds(bprev * 16, 16)] = m

        pltpu.sync_copy(out_v, out_ref.at[w, pl.ds(j * SEGP * 16, SEGP * 16)])
        return 0

    lax.fori_loop(0, nf, chunk_body, 0)


def _sc_segmax(s3, dis1d, b3, batch_pad):
    nf = s3.shape[1] // 16
    mesh = plsc.VectorSubcoreMesh(core_axis_name="c", subcore_axis_name="s")
    return pl.kernel(
        functools.partial(_segmax_body, nf),
        out_type=jax.ShapeDtypeStruct((NW, nf * SEGP * 16), jnp.float32),
        mesh=mesh,
        compiler_params=pltpu.CompilerParams(use_tc_tiling_on_sc=False),
        scratch_types=[
            pltpu.VMEM((WR, 16), jnp.float32),
            pltpu.VMEM((WR, 16), jnp.float32),
            pltpu.VMEM((RW + 16,), jnp.int32),
            pltpu.VMEM((RW + 16,), jnp.float32),
            pltpu.VMEM((nf * 16,), jnp.float32),
            pltpu.VMEM((SEGP * 16,), jnp.float32),
            pltpu.SemaphoreType.DMA,
            pltpu.SemaphoreType.DMA,
        ],
    )(s3, dis1d, b3, batch_pad)


def _dis_body(p_ref, o_ref):
    o_ref[...] = 1.0 / jnp.sqrt(1.0 + p_ref[0] + p_ref[1])


def _tc_dis(partials):
    p = partials.reshape(NC, NP // 128, 128)
    out = pl.pallas_call(
        _dis_body,
        out_shape=jax.ShapeDtypeStruct((NP // 128, 128), jnp.float32),
    )(p)
    return out.reshape(NP, 1)


def _l1_body(x_ref, w_ref, dis_ref, o_ref):
    o_ref[...] = (jnp.dot(x_ref[...], w_ref[0],
                          preferred_element_type=jnp.float32)
                  * dis_ref[...])[None]


def _tc_layer1(x_pad, W, dis):
    k = x_pad.shape[1]
    nf = W.shape[1] // 16
    w_r = W.reshape(k, nf, 16).transpose(1, 0, 2)
    return pl.pallas_call(
        _l1_body,
        grid=(N_RB, nf),
        in_specs=[
            pl.BlockSpec((R, k), lambda i, j: (i, 0)),
            pl.BlockSpec((1, k, 16), lambda i, j: (j, 0, 0)),
            pl.BlockSpec((R, 1), lambda i, j: (i, 0)),
        ],
        out_specs=pl.BlockSpec((1, R, 16), lambda i, j: (j, i, 0)),
        out_shape=jax.ShapeDtypeStruct((nf, NP, 16), jnp.float32),
    )(x_pad, w_r, dis)


def _mid_body(s_ref, w_ref, dis_ref, b_ref, o_ref, h_scr):
    @pl.when(pl.program_id(1) == 0)
    def _():
        h_scr[...] = jnp.maximum(s_ref[...] * dis_ref[...] + b_ref[...], 0.0)

    o_ref[...] = (jnp.dot(h_scr[...], w_ref[0],
                          preferred_element_type=jnp.float32)
                  * dis_ref[...])[None]


def _tc_mid(s_prev, W, dis, b_prev):
    fin = s_prev.shape[1]
    nf = W.shape[1] // 16
    w_r = W.reshape(fin, nf, 16).transpose(1, 0, 2)
    return pl.pallas_call(
        _mid_body,
        grid=(N_RB, nf),
        in_specs=[
            pl.BlockSpec((R, fin), lambda i, j: (i, 0)),
            pl.BlockSpec((1, fin, 16), lambda i, j: (j, 0, 0)),
            pl.BlockSpec((R, 1), lambda i, j: (i, 0)),
            pl.BlockSpec((1, fin), lambda i, j: (0, 0)),
        ],
        out_specs=pl.BlockSpec((1, R, 16), lambda i, j: (j, i, 0)),
        out_shape=jax.ShapeDtypeStruct((nf, NP, 16), jnp.float32),
        scratch_shapes=[pltpu.VMEM((R, fin), jnp.float32)],
    )(s_prev, w_r, dis, b_prev)


def _gmax_body(nf, p_ref, o_ref):
    @pl.when(pl.program_id(0) == 0)
    def _():
        o_ref[...] = jnp.full_like(o_ref, -1e30)
    g = jnp.concatenate([p_ref[0, j] for j in range(nf)], axis=1)
    o_ref[...] = jnp.maximum(o_ref[...], g)


def _tc_gmax(partials_flat, nf):
    p = partials_flat.reshape(NW, nf, SEGP, 16)
    return pl.pallas_call(
        functools.partial(_gmax_body, nf),
        grid=(NW,),
        in_specs=[pl.BlockSpec((1, nf, SEGP, 16), lambda i: (i, 0, 0, 0))],
        out_specs=pl.BlockSpec((SEGP, nf * 16), lambda i: (0, 0)),
        out_shape=jax.ShapeDtypeStruct((SEGP, nf * 16), jnp.float32),
    )(p)


def _head_body(g_ref, tp_ref, wg1_ref, bg1_ref, wg2_ref, bg2_ref,
               wf1_ref, bf1_ref, wf2_ref, bf2_ref, wo_ref, bo_ref, out_ref):
    g = g_ref[...]
    h = jnp.maximum(jnp.dot(g, wg1_ref[...], preferred_element_type=jnp.float32)
                    + bg1_ref[...], 0.0)
    g2 = jnp.dot(h, wg2_ref[...], preferred_element_type=jnp.float32) + bg2_ref[...]
    xc = jnp.concatenate([g2, tp_ref[...]], axis=1)
    f1 = jnp.maximum(jnp.dot(xc, wf1_ref[...], preferred_element_type=jnp.float32)
                     + bf1_ref[...], 0.0)
    f2 = jnp.maximum(jnp.dot(f1, wf2_ref[...], preferred_element_type=jnp.float32)
                     + bf2_ref[...], 0.0)
    out_ref[...] = jnp.dot(f2, wo_ref[...], preferred_element_type=jnp.float32) + bo_ref[...]


def _mlp_head(g, T, P, Wg1, bg1, Wg2, bg2, Wf1, bf1, Wf2, bf2, Wo, bo):
    B = T.shape[0]
    F = Wg1.shape[0]
    Fp = g.shape[1]
    wg1 = jnp.pad(Wg1, ((0, Fp - F), (0, 0)))
    tp = jnp.pad(jnp.stack([T, P], axis=1), ((0, 0), (0, 126)))
    wf1 = jnp.pad(Wf1, ((0, 256 - Wf1.shape[0]), (0, 0)))
    wo = jnp.pad(Wo, ((0, 0), (0, 127)))
    bo_p = jnp.pad(bo, ((0, 127)))
    out = pl.pallas_call(
        _head_body,
        out_shape=jax.ShapeDtypeStruct((B, 128), jnp.float32),
    )(g, tp, wg1, bg1.reshape(1, -1), Wg2, bg2.reshape(1, -1),
      wf1, bf1.reshape(1, -1), Wf2, bf2.reshape(1, -1), wo, bo_p.reshape(1, -1))
    return out[:, :1]


def _pad_w(W, b):
    fin, fout = W.shape
    fi = ((fin + 15) // 16) * 16
    fo = ((fout + 15) // 16) * 16
    return (jnp.pad(W, ((0, fi - fin), (0, fo - fout))),
            jnp.pad(b, (0, fo - fout)).reshape(1, fo))


def kernel(x, edge_index, batch, T, P, W1, b1, W2, b2, W3, b3,
           Wg1, bg1, Wg2, bg2, Wf1, bf1, Wf2, bf2, Wo, bo):
    n = x.shape[0]
    B = T.shape[0]
    src = edge_index[0]
    dst = edge_index[1]

    partials = _sc_deg(dst)
    dis = _tc_dis(partials)

    w1, b1p = _pad_w(W1, b1)
    x_pad = jnp.pad(x, ((0, NP - n), (0, w1.shape[0] - x.shape[1])))
    w2, b2p = _pad_w(W2, b2)
    w3, b3p = _pad_w(W3, b3)
    batch_pad = jnp.concatenate([batch, jnp.full((NP - n,), B, jnp.int32)])

    iota = jnp.arange(n, dtype=jnp.int32)
    pad_e = n + jnp.arange(E_EXT - E_TOT - n, dtype=jnp.int32) % (NP - n)
    src_ext = jnp.concatenate([src, iota, pad_e])
    dst_ext = jnp.concatenate([dst, iota, pad_e])
    zeros_hbm = jnp.zeros((NP, 16), jnp.float32)

    y1 = _tc_layer1(x_pad, w1, dis)
    s1 = _sc_scatter(y1, zeros_hbm, src_ext, dst_ext)
    y2 = _tc_mid(s1, w2, dis, b1p)
    s2 = _sc_scatter(y2, zeros_hbm, src_ext, dst_ext)
    y3 = _tc_mid(s2, w3, dis, b2p)
    s3 = _sc_scatter(y3, zeros_hbm, src_ext, dst_ext)

    partials_g = _sc_segmax(s3, dis.reshape(NP), b3p.reshape(-1), batch_pad)
    g = _tc_gmax(partials_g, w3.shape[1] // 16)
    return _mlp_head(g[:B, :], T, P, Wg1, bg1, Wg2, bg2, Wf1, bf1, Wf2, bf2, Wo, bo)
